# baseline (speedup 1.0000x reference)
import jax
import jax.numpy as jnp
from jax.experimental import pallas as pl
from jax.experimental.pallas import tpu as pltpu

TAU_ = 0.8
LAMBDA_ = 0.5
LN2_ = 0.6931471805599453


def _make_loss_kernel(n, npad, b, rb):
    def _projnorm(z, w1t, b1, w2t, b2):
        h = jnp.dot(z, w1t, preferred_element_type=jnp.float32) + b1
        h = jnp.where(h > 0, h, jnp.exp(h) - 1.0)
        p = jnp.dot(h, w2t, preferred_element_type=jnp.float32) + b2
        norm2 = jnp.dot(p * p, jnp.ones((p.shape[1], 1), jnp.float32),
                        preferred_element_type=jnp.float32)
        return p * jax.lax.rsqrt(norm2)

    def _loss_kernel(zsc_ref, zmp_ref, w1t_ref, b1_ref, w2t_ref, b2_ref,
                     pos_ref, out_ref, khs_ref, khm_ref):
        i = pl.program_id(0)

        @pl.when(i == 0)
        def _init():
            w1t, b1 = w1t_ref[...], b1_ref[...]
            w2t, b2 = w2t_ref[...], b2_ref[...]
            scale = jnp.float32(1.0 / (TAU_ * LN2_))
            zh_sc = _projnorm(zsc_ref[...], w1t, b1, w2t, b2) * scale
            zh_mp = _projnorm(zmp_ref[...], w1t, b1, w2t, b2)
            khs_ref[pl.ds(0, n), :] = zh_sc.astype(jnp.bfloat16)
            khm_ref[pl.ds(0, n), :] = zh_mp.astype(jnp.bfloat16)
            pad16 = jnp.zeros((npad - n, khs_ref.shape[1]), jnp.bfloat16)
            khs_ref[pl.ds(n, npad - n), :] = pad16
            khm_ref[pl.ds(n, npad - n), :] = pad16
            out_ref[...] = jnp.zeros((1, 1), jnp.float32)

        qs = khs_ref[pl.ds(i * rb, rb), :]
        qm = khm_ref[pl.ds(i * rb, rb), :]
        psel = pos_ref[...] != 0
        zero = jnp.zeros((), jnp.float32)

        dn = (((1,), (1,)), ((), ()))
        pad = jnp.float32(npad - n)
        eps = jnp.float32(1e-8)

        e1 = jnp.exp2(jax.lax.dot_general(qs, khm_ref[...], dn,
                                          preferred_element_type=jnp.float32))
        num1 = jnp.sum(jnp.where(psel, e1[:, :n], zero), axis=1, keepdims=True)
        den1 = jnp.sum(e1, axis=1, keepdims=True) - pad
        e2 = jnp.exp2(jax.lax.dot_general(qm, khs_ref[...], dn,
                                          preferred_element_type=jnp.float32))
        num2 = jnp.sum(jnp.where(psel, e2[:, :n], zero), axis=1, keepdims=True)
        den2 = jnp.sum(e2, axis=1, keepdims=True) - pad

        c1 = jnp.float32(-LAMBDA_ / b)
        c2 = jnp.float32(-(1.0 - LAMBDA_) / b)
        part = (c1 * jnp.sum(jnp.log(num1 / (den1 + eps)))
                + c2 * jnp.sum(jnp.log(num2 / (den2 + eps))))
        out_ref[...] += jnp.reshape(part, (1, 1))

    return _loss_kernel


def kernel(z_sc, z_mp, pos, W1, b1, W2, b2):
    N, d = z_sc.shape
    B = pos.shape[0]
    Npad = -(-N // 128) * 128
    RB = 128
    n_blocks = B // RB

    out = pl.pallas_call(
        _make_loss_kernel(N, Npad, B, RB),
        grid=(n_blocks,),
        in_specs=[
            pl.BlockSpec((N, d), lambda i: (0, 0)),
            pl.BlockSpec((N, d), lambda i: (0, 0)),
            pl.BlockSpec((d, d), lambda i: (0, 0)),
            pl.BlockSpec((1, d), lambda i: (0, 0)),
            pl.BlockSpec((d, d), lambda i: (0, 0)),
            pl.BlockSpec((1, d), lambda i: (0, 0)),
            pl.BlockSpec((RB, N), lambda i: (i, 0)),
        ],
        out_specs=pl.BlockSpec((1, 1), lambda i: (0, 0)),
        out_shape=jax.ShapeDtypeStruct((1, 1), jnp.float32),
        scratch_shapes=[
            pltpu.VMEM((Npad, d), jnp.bfloat16),
            pltpu.VMEM((Npad, d), jnp.bfloat16),
        ],
    )(z_sc, z_mp, W1.T, b1.reshape(1, d), W2.T, b2.reshape(1, d),
      pos.astype(jnp.int8))

    return out[0, 0]


# speedup vs baseline: 1.4245x; 1.4245x over previous
import jax
import jax.numpy as jnp
from jax.experimental import pallas as pl
from jax.experimental.pallas import tpu as pltpu

TAU_ = 0.8
LAMBDA_ = 0.5
LN2_ = 0.6931471805599453


def _make_loss_kernel(n, npad, b, rb):
    def _projnorm(z, w1t, b1, w2t, b2):
        h = jnp.dot(z, w1t, preferred_element_type=jnp.float32) + b1
        h = jnp.where(h > 0, h, jnp.exp(h) - 1.0)
        p = jnp.dot(h, w2t, preferred_element_type=jnp.float32) + b2
        norm2 = jnp.dot(p * p, jnp.ones((p.shape[1], 1), jnp.float32),
                        preferred_element_type=jnp.float32)
        return p * jax.lax.rsqrt(norm2)

    def _loss_kernel(zsc_ref, zmp_ref, w1t_ref, b1_ref, w2t_ref, b2_ref,
                     pos_ref, out_ref, khs_ref, khm_ref):
        i = pl.program_id(0)

        @pl.when(i == 0)
        def _init():
            w1t, b1 = w1t_ref[...], b1_ref[...]
            w2t, b2 = w2t_ref[...], b2_ref[...]
            scale = jnp.float32(1.0 / (TAU_ * LN2_))
            zh_sc = _projnorm(zsc_ref[...], w1t, b1, w2t, b2) * scale
            zh_mp = _projnorm(zmp_ref[...], w1t, b1, w2t, b2)
            khs_ref[pl.ds(0, n), :] = zh_sc.astype(jnp.bfloat16)
            khm_ref[pl.ds(0, n), :] = zh_mp.astype(jnp.bfloat16)
            pad16 = jnp.zeros((npad - n, khs_ref.shape[1]), jnp.bfloat16)
            khs_ref[pl.ds(n, npad - n), :] = pad16
            khm_ref[pl.ds(n, npad - n), :] = pad16
            out_ref[...] = jnp.zeros((1, 1), jnp.float32)

        qs = khs_ref[pl.ds(i * rb, rb), :]
        qm = khm_ref[pl.ds(i * rb, rb), :]
        p = pos_ref[...].astype(jnp.float32)

        dn = (((1,), (1,)), ((), ()))
        pad = jnp.float32(npad - n)
        eps = jnp.float32(1e-8)

        e1 = jnp.exp2(jax.lax.dot_general(khm_ref[...], qs, dn,
                                          preferred_element_type=jnp.float32))
        num1 = jnp.sum(e1[:n, :] * p, axis=0, keepdims=True)
        den1 = jnp.sum(e1, axis=0, keepdims=True) - pad
        e2 = jnp.exp2(jax.lax.dot_general(khs_ref[...], qm, dn,
                                          preferred_element_type=jnp.float32))
        num2 = jnp.sum(e2[:n, :] * p, axis=0, keepdims=True)
        den2 = jnp.sum(e2, axis=0, keepdims=True) - pad

        c1 = jnp.float32(-LAMBDA_ / b)
        c2 = jnp.float32(-(1.0 - LAMBDA_) / b)
        part = (c1 * jnp.sum(jnp.log(num1 / (den1 + eps)))
                + c2 * jnp.sum(jnp.log(num2 / (den2 + eps))))
        out_ref[...] += jnp.reshape(part, (1, 1))

    return _loss_kernel


def kernel(z_sc, z_mp, pos, W1, b1, W2, b2):
    N, d = z_sc.shape
    B = pos.shape[0]
    Npad = -(-N // 128) * 128
    RB = 128
    n_blocks = B // RB

    out = pl.pallas_call(
        _make_loss_kernel(N, Npad, B, RB),
        grid=(n_blocks,),
        in_specs=[
            pl.BlockSpec((N, d), lambda i: (0, 0)),
            pl.BlockSpec((N, d), lambda i: (0, 0)),
            pl.BlockSpec((d, d), lambda i: (0, 0)),
            pl.BlockSpec((1, d), lambda i: (0, 0)),
            pl.BlockSpec((d, d), lambda i: (0, 0)),
            pl.BlockSpec((1, d), lambda i: (0, 0)),
            pl.BlockSpec((N, RB), lambda i: (0, i)),
        ],
        out_specs=pl.BlockSpec((1, 1), lambda i: (0, 0)),
        out_shape=jax.ShapeDtypeStruct((1, 1), jnp.float32),
        scratch_shapes=[
            pltpu.VMEM((Npad, d), jnp.bfloat16),
            pltpu.VMEM((Npad, d), jnp.bfloat16),
        ],
    )(z_sc, z_mp, W1.T, b1.reshape(1, d), W2.T, b2.reshape(1, d),
      pos.astype(jnp.int8).T)

    return out[0, 0]
